# baseline (speedup 1.0000x reference)
import jax
import jax.numpy as jnp
from jax.experimental import pallas as pl
from jax.experimental.pallas import tpu as pltpu

DIM = 128
QUEUE_SIZE = 65536
BATCH_COLS = 4096

_W = 4096
_NCHUNK = QUEUE_SIZE // _W
_NBUF = 4


def _ring_body(lk_ref, q_ref, out_ref, buf, rsem, wsem):
    def read(c):
        b = c % _NBUF
        if c == 0:
            src = lk_ref
        else:
            src = q_ref.at[:, pl.ds(c * _W, _W)]
        pltpu.make_async_copy(src, buf.at[b], rsem.at[b]).start()

    def wait_read(c):
        b = c % _NBUF
        if c == 0:
            src = lk_ref
        else:
            src = q_ref.at[:, pl.ds(c * _W, _W)]
        pltpu.make_async_copy(src, buf.at[b], rsem.at[b]).wait()

    def write(c):
        b = c % _NBUF
        pltpu.make_async_copy(buf.at[b], out_ref.at[:, pl.ds(c * _W, _W)], wsem.at[b]).start()

    def wait_write(c):
        b = c % _NBUF
        pltpu.make_async_copy(buf.at[b], out_ref.at[:, pl.ds(c * _W, _W)], wsem.at[b]).wait()

    for c in range(_NBUF):
        read(c)
    for c in range(_NCHUNK):
        wait_read(c)
        write(c)
        if c + _NBUF < _NCHUNK:
            wait_write(c)
            read(c + _NBUF)
    for c in range(_NCHUNK - _NBUF, _NCHUNK):
        wait_write(c)


def kernel(last_k, moco_queue):
    return pl.pallas_call(
        _ring_body,
        in_specs=[
            pl.BlockSpec(memory_space=pl.ANY),
            pl.BlockSpec(memory_space=pl.ANY),
        ],
        out_specs=pl.BlockSpec(memory_space=pl.ANY),
        out_shape=jax.ShapeDtypeStruct((DIM, QUEUE_SIZE), jnp.float32),
        scratch_shapes=[
            pltpu.VMEM((_NBUF, DIM, _W), jnp.float32),
            pltpu.SemaphoreType.DMA((_NBUF,)),
            pltpu.SemaphoreType.DMA((_NBUF,)),
        ],
    )(last_k, moco_queue)


# speedup vs baseline: 1.1705x; 1.1705x over previous
import jax
import jax.numpy as jnp
from jax.experimental import pallas as pl
from jax.experimental.pallas import tpu as pltpu

DIM = 128
QUEUE_SIZE = 65536
BATCH_COLS = 4096

_W = 8192
_NCHUNK = QUEUE_SIZE // _W
_NBUF = 6


def _ring_body(lk_ref, q_ref, out_ref, buf, rsem, wsem):
    def read_descs(c):
        b = c % _NBUF
        if c == 0:
            return [
                pltpu.make_async_copy(lk_ref, buf.at[b, :, pl.ds(0, BATCH_COLS)], rsem.at[b]),
                pltpu.make_async_copy(
                    q_ref.at[:, pl.ds(BATCH_COLS, _W - BATCH_COLS)],
                    buf.at[b, :, pl.ds(BATCH_COLS, _W - BATCH_COLS)],
                    rsem.at[b],
                ),
            ]
        return [
            pltpu.make_async_copy(
                q_ref.at[:, pl.ds(c * _W, _W)], buf.at[b], rsem.at[b]
            )
        ]

    def write_desc(c):
        b = c % _NBUF
        return pltpu.make_async_copy(
            buf.at[b], out_ref.at[:, pl.ds(c * _W, _W)], wsem.at[b]
        )

    for c in range(_NBUF):
        for d in read_descs(c):
            d.start()
    for c in range(_NCHUNK):
        for d in read_descs(c):
            d.wait()
        write_desc(c).start()
        if c + _NBUF < _NCHUNK:
            write_desc(c).wait()
            for d in read_descs(c + _NBUF):
                d.start()
    for c in range(max(_NCHUNK - _NBUF, 0), _NCHUNK):
        write_desc(c).wait()


def kernel(last_k, moco_queue):
    return pl.pallas_call(
        _ring_body,
        in_specs=[
            pl.BlockSpec(memory_space=pl.ANY),
            pl.BlockSpec(memory_space=pl.ANY),
        ],
        out_specs=pl.BlockSpec(memory_space=pl.ANY),
        out_shape=jax.ShapeDtypeStruct((DIM, QUEUE_SIZE), jnp.float32),
        scratch_shapes=[
            pltpu.VMEM((_NBUF, DIM, _W), jnp.float32),
            pltpu.SemaphoreType.DMA((_NBUF,)),
            pltpu.SemaphoreType.DMA((_NBUF,)),
        ],
    )(last_k, moco_queue)
